# baseline (speedup 1.0000x reference)
import functools

import jax
import jax.numpy as jnp
from jax import lax
from jax.experimental import pallas as pl
from jax.experimental.pallas import tpu as pltpu
from jax.experimental.pallas import tpu_sc as plsc

_RES = 128
_N_PTS = 4194304
_N_WORDS = 65536

_NW = 32
_PTS_PER_W = _N_PTS // _NW
_CHUNK = 4096
_N_CHUNKS = _PTS_PER_W // _CHUNK
_VECS = _CHUNK // 16
_LIN_BIAS = -133160832


def _pack_body(g_ref, out_ref):
    g = g_ref[...].reshape(32, 4, 128, 128).astype(jnp.int32)
    shifts = lax.broadcasted_iota(jnp.int32, (32, 1, 1, 1), 0)
    out_ref[...] = jnp.sum(g << shifts, axis=0)


_pack_grid = pl.pallas_call(
    _pack_body,
    out_shape=jax.ShapeDtypeStruct((4, 128, 128), jnp.int32),
)


def _make_sc_kernel(n_pts):
    pts_per_w = n_pts // _NW
    n_chunks = pts_per_w // _CHUNK
    mesh = plsc.VectorSubcoreMesh(core_axis_name="c", subcore_axis_name="s")

    @functools.partial(
        pl.kernel,
        mesh=mesh,
        compiler_params=pltpu.CompilerParams(needs_layout_passes=False),
        out_type=jax.ShapeDtypeStruct((n_pts,), jnp.int32),
        scratch_types=[
            pltpu.VMEM((_N_WORDS,), jnp.int32),
            pltpu.VMEM((_CHUNK,), jnp.float32),
            pltpu.VMEM((_CHUNK,), jnp.float32),
            pltpu.VMEM((_CHUNK,), jnp.float32),
            pltpu.VMEM((_CHUNK,), jnp.float32),
            pltpu.VMEM((_CHUNK,), jnp.float32),
            pltpu.VMEM((_CHUNK,), jnp.float32),
            pltpu.VMEM((_CHUNK,), jnp.int32),
            pltpu.VMEM((_CHUNK,), jnp.int32),
            pltpu.SemaphoreType.DMA,
            pltpu.SemaphoreType.DMA,
            pltpu.SemaphoreType.DMA,
            pltpu.SemaphoreType.DMA,
        ],
    )
    def sc_kernel(xs_hbm, ys_hbm, zs_hbm, grid_hbm, out_hbm, grid_v, xs0, ys0,
                  zs0, xs1, ys1, zs1, out0, out1, sin0, sin1, sout0, sout1):
        wid = lax.axis_index("s") * 2 + lax.axis_index("c")
        base = wid * pts_per_w
        pltpu.sync_copy(grid_hbm, grid_v)
        inb = ((xs0, ys0, zs0, sin0), (xs1, ys1, zs1, sin1))
        outb = ((out0, sout0), (out1, sout1))

        def start_in(c, b):
            xs, ys, zs, sem = inb[b]
            off = base + c * _CHUNK
            pltpu.async_copy(xs_hbm.at[pl.ds(off, _CHUNK)], xs, sem)
            pltpu.async_copy(ys_hbm.at[pl.ds(off, _CHUNK)], ys, sem)
            pltpu.async_copy(zs_hbm.at[pl.ds(off, _CHUNK)], zs, sem)

        def wait_in(b):
            xs, ys, zs, sem = inb[b]
            pltpu.make_async_copy(xs_hbm.at[pl.ds(0, _CHUNK)], xs, sem).wait()
            pltpu.make_async_copy(ys_hbm.at[pl.ds(0, _CHUNK)], ys, sem).wait()
            pltpu.make_async_copy(zs_hbm.at[pl.ds(0, _CHUNK)], zs, sem).wait()

        def start_out(c, b):
            out_v, sem = outb[b]
            pltpu.async_copy(out_v, out_hbm.at[pl.ds(base + c * _CHUNK, _CHUNK)],
                             sem)

        def wait_out(b):
            out_v, sem = outb[b]
            pltpu.make_async_copy(out_v, out_hbm.at[pl.ds(0, _CHUNK)],
                                  sem).wait()

        def compute(b):
            xs, ys, zs, _ = inb[b]
            out_v, _ = outb[b]

            @plsc.parallel_loop(0, _VECS, unroll=16)
            def vec_body(j):
                s = pl.ds(j * 16, 16)
                sx = plsc.bitcast(xs[s] + 1.0, jnp.int32) >> 17
                sy = plsc.bitcast(ys[s] + 1.0, jnp.int32) >> 17
                sz = plsc.bitcast(zs[s] + 1.0, jnp.int32) >> 17
                valid = (sx | sy | sz) < 8192
                lin = sx * 16384 + sy * 128 + sz + _LIN_BIAS
                w = plsc.load_gather(grid_v, [lin & 0xFFFF])
                val = (w >> (lin >> 16)) & 1
                out_v[s] = jnp.where(valid, val, 0)

        start_in(0, 0)
        start_in(1, 1)

        def c2_body(c2, carry):
            for b in (0, 1):
                c = c2 * 2 + b
                wait_in(b)

                @pl.when(c2 > 0)
                def _():
                    wait_out(b)

                compute(b)
                start_out(c, b)

                @pl.when(c < n_chunks - 2)
                def _():
                    start_in(c + 2, b)
            return carry

        lax.fori_loop(0, n_chunks // 2, c2_body, 0)
        wait_out(0)
        wait_out(1)

    return sc_kernel


_sc_gather = _make_sc_kernel(_N_PTS)


def kernel(pts, binary_grid):
    words = _pack_grid(binary_grid).reshape(-1)
    out = _sc_gather(pts[:, 0], pts[:, 1], pts[:, 2], words)
    return out.astype(jnp.bool_)


# speedup vs baseline: 1.0253x; 1.0253x over previous
import functools

import jax
import jax.numpy as jnp
from jax import lax
from jax.experimental import pallas as pl
from jax.experimental.pallas import tpu as pltpu
from jax.experimental.pallas import tpu_sc as plsc

_RES = 128
_N_PTS = 4194304
_N_WORDS = 65536

_NW = 32
_PTS_PER_W = _N_PTS // _NW
_CHUNK = 4096
_N_CHUNKS = _PTS_PER_W // _CHUNK
_VECS = _CHUNK // 16
_LIN_BIAS = -133160832


def _pack_body(g_ref, out_ref):
    g = g_ref[...].reshape(32, 4, 128, 128).astype(jnp.int32)
    shifts = lax.broadcasted_iota(jnp.int32, (32, 1, 1, 1), 0)
    out_ref[...] = jnp.sum(g << shifts, axis=0)


_pack_grid = pl.pallas_call(
    _pack_body,
    out_shape=jax.ShapeDtypeStruct((4, 128, 128), jnp.int32),
)


def _make_sc_kernel(n_pts):
    pts_per_w = n_pts // _NW
    n_chunks = pts_per_w // _CHUNK
    mesh = plsc.VectorSubcoreMesh(core_axis_name="c", subcore_axis_name="s")

    @functools.partial(
        pl.kernel,
        mesh=mesh,
        compiler_params=pltpu.CompilerParams(needs_layout_passes=False),
        out_type=jax.ShapeDtypeStruct((n_pts,), jnp.int32),
        scratch_types=[
            pltpu.VMEM((_N_WORDS,), jnp.int32),
            pltpu.VMEM((_CHUNK,), jnp.float32),
            pltpu.VMEM((_CHUNK,), jnp.float32),
            pltpu.VMEM((_CHUNK,), jnp.float32),
            pltpu.VMEM((_CHUNK,), jnp.float32),
            pltpu.VMEM((_CHUNK,), jnp.float32),
            pltpu.VMEM((_CHUNK,), jnp.float32),
            pltpu.VMEM((_CHUNK,), jnp.int32),
            pltpu.VMEM((_CHUNK,), jnp.int32),
            pltpu.SemaphoreType.DMA,
            pltpu.SemaphoreType.DMA,
            pltpu.SemaphoreType.DMA,
            pltpu.SemaphoreType.DMA,
        ],
    )
    def sc_kernel(xs_hbm, ys_hbm, zs_hbm, grid_hbm, out_hbm, grid_v, xs0, ys0,
                  zs0, xs1, ys1, zs1, out0, out1, sin0, sin1, sout0, sout1):
        wid = lax.axis_index("s") * 2 + lax.axis_index("c")
        base = wid * pts_per_w
        pltpu.sync_copy(grid_hbm, grid_v)
        inb = ((xs0, ys0, zs0, sin0), (xs1, ys1, zs1, sin1))
        outb = ((out0, sout0), (out1, sout1))

        def start_in(c, b):
            xs, ys, zs, sem = inb[b]
            off = base + c * _CHUNK
            pltpu.async_copy(xs_hbm.at[pl.ds(off, _CHUNK)], xs, sem)
            pltpu.async_copy(ys_hbm.at[pl.ds(off, _CHUNK)], ys, sem)
            pltpu.async_copy(zs_hbm.at[pl.ds(off, _CHUNK)], zs, sem)

        def wait_in(b):
            xs, ys, zs, sem = inb[b]
            pltpu.make_async_copy(xs_hbm.at[pl.ds(0, _CHUNK)], xs, sem).wait()
            pltpu.make_async_copy(ys_hbm.at[pl.ds(0, _CHUNK)], ys, sem).wait()
            pltpu.make_async_copy(zs_hbm.at[pl.ds(0, _CHUNK)], zs, sem).wait()

        def start_out(c, b):
            out_v, sem = outb[b]
            pltpu.async_copy(out_v, out_hbm.at[pl.ds(base + c * _CHUNK, _CHUNK)],
                             sem)

        def wait_out(b):
            out_v, sem = outb[b]
            pltpu.make_async_copy(out_v, out_hbm.at[pl.ds(0, _CHUNK)],
                                  sem).wait()

        def compute(b):
            xs, ys, zs, _ = inb[b]
            out_v, _ = outb[b]

            @plsc.parallel_loop(0, _VECS, unroll=8)
            def vec_body(j):
                s = pl.ds(j * 16, 16)
                sx = plsc.bitcast(xs[s] + 1.0, jnp.int32) >> 17
                sy = plsc.bitcast(ys[s] + 1.0, jnp.int32) >> 17
                sz = plsc.bitcast(zs[s] + 1.0, jnp.int32) >> 17
                valid = (sx | sy | sz) < 8192
                lin = sx * 16384 + sy * 128 + sz + _LIN_BIAS
                w = plsc.load_gather(grid_v, [lin & 0xFFFF])
                val = (w >> (lin >> 16)) & 1
                out_v[s] = jnp.where(valid, val, 0)

        start_in(0, 0)
        start_in(1, 1)

        def c2_body(c2, carry):
            for b in (0, 1):
                c = c2 * 2 + b
                wait_in(b)

                @pl.when(c2 > 0)
                def _():
                    wait_out(b)

                compute(b)
                start_out(c, b)

                @pl.when(c < n_chunks - 2)
                def _():
                    start_in(c + 2, b)
            return carry

        lax.fori_loop(0, n_chunks // 2, c2_body, 0)
        wait_out(0)
        wait_out(1)

    return sc_kernel


_sc_gather = _make_sc_kernel(_N_PTS)


def kernel(pts, binary_grid):
    words = _pack_grid(binary_grid).reshape(-1)
    out = _sc_gather(pts[:, 0], pts[:, 1], pts[:, 2], words)
    return out.astype(jnp.bool_)
